# Initial kernel scaffold; baseline (speedup 1.0000x reference)
import jax
import jax.numpy as jnp
from jax.experimental import pallas as pl


def kernel(inputs, word_table, pos_table):
    raise NotImplementedError("write your pallas kernel here")


# speedup vs baseline: 1.2528x; 1.2528x over previous
import functools

import jax
import jax.numpy as jnp
from jax import lax
from jax.experimental import pallas as pl
from jax.experimental.pallas import tpu as pltpu
from jax.experimental.pallas import tpu_sc as plsc

SEQ = 200
DIM = 32
NW = 32
CHUNK = 400
GSUB = 80
NGS = CHUNK // GSUB


def _sc_body(idx_hbm, word_hbm, pos_hbm, out_hbm, idx_v, rows_v, pos_v, sem):
    nc = 2
    wid = lax.axis_index("s") * nc + lax.axis_index("c")
    n_rows = idx_hbm.shape[0]
    per_w = n_rows // NW
    n_chunks = per_w // CHUNK
    base = wid * per_w

    pltpu.sync_copy(pos_hbm, pos_v.at[pl.ds(0, SEQ)])
    pltpu.sync_copy(pos_hbm, pos_v.at[pl.ds(SEQ, SEQ)])

    def chunk_body(ci, _):
        row0 = base + ci * CHUNK
        pltpu.sync_copy(idx_hbm.at[pl.ds(row0, CHUNK)], idx_v)
        copies = []
        for j in range(NGS):
            copies.append(pltpu.async_copy(
                word_hbm.at[idx_v.at[pl.ds(j * GSUB, GSUB)]],
                rows_v.at[pl.ds(j * GSUB, GSUB)],
                sem,
            ))
        for c in copies:
            c.wait()

        def add_body(r, _):
            rows_v[r, pl.ds(0, 16)] = rows_v[r, pl.ds(0, 16)] + pos_v[r, pl.ds(0, 16)]
            rows_v[r, pl.ds(16, 16)] = rows_v[r, pl.ds(16, 16)] + pos_v[r, pl.ds(16, 16)]
            return _

        lax.fori_loop(0, CHUNK, add_body, None)
        pltpu.sync_copy(rows_v, out_hbm.at[pl.ds(row0, CHUNK)])
        return _

    lax.fori_loop(0, n_chunks, chunk_body, None)


def kernel(inputs, word_table, pos_table):
    b, s = inputs.shape
    n = b * s
    idx_flat = inputs.reshape(n).astype(jnp.int32)
    mesh = plsc.VectorSubcoreMesh(core_axis_name="c", subcore_axis_name="s")
    out = pl.kernel(
        _sc_body,
        out_type=jax.ShapeDtypeStruct((n, DIM), jnp.float32),
        mesh=mesh,
        compiler_params=pltpu.CompilerParams(use_tc_tiling_on_sc=False),
        scratch_types=[
            pltpu.VMEM((CHUNK,), jnp.int32),
            pltpu.VMEM((CHUNK, DIM), jnp.float32),
            pltpu.VMEM((CHUNK, DIM), jnp.float32),
            pltpu.SemaphoreType.DMA,
        ],
    )(idx_flat, word_table, pos_table)
    return out.reshape(b, s, DIM)


# speedup vs baseline: 1.4610x; 1.1662x over previous
import jax
import jax.numpy as jnp
from jax import lax
from jax.experimental import pallas as pl
from jax.experimental.pallas import tpu as pltpu
from jax.experimental.pallas import tpu_sc as plsc

SEQ = 200
DIM = 32
NW = 32
CHUNK = 400
GSUB = 80
NGS = CHUNK // GSUB


def _sc_body(idx_hbm, word_hbm, pos_hbm, out_hbm,
             idx_v, rows_v, pos_v, idx_sem, gat_sem, out_sem):
    nc = 2
    wid = lax.axis_index("s") * nc + lax.axis_index("c")
    n_rows = idx_hbm.shape[0]
    per_w = n_rows // NW
    n_chunks = per_w // CHUNK
    base = wid * per_w

    pltpu.sync_copy(pos_hbm, pos_v.at[pl.ds(0, SEQ)])
    pltpu.sync_copy(pos_hbm, pos_v.at[pl.ds(SEQ, SEQ)])

    def idx_copy(ci, b):
        return pltpu.make_async_copy(
            idx_hbm.at[pl.ds(base + ci * CHUNK, CHUNK)],
            idx_v.at[b], idx_sem.at[b])

    def gat_copy(b, j):
        return pltpu.make_async_copy(
            word_hbm.at[idx_v.at[b, pl.ds(j * GSUB, GSUB)]],
            rows_v.at[b, pl.ds(j * GSUB, GSUB)], gat_sem.at[b])

    def out_copy(ci, b):
        return pltpu.make_async_copy(
            rows_v.at[b], out_hbm.at[pl.ds(base + ci * CHUNK, CHUNK)],
            out_sem.at[b])

    idx_copy(0, 0).start()
    idx_copy(1, 1).start()
    idx_copy(0, 0).wait()
    for j in range(NGS):
        gat_copy(0, j).start()

    def pair_body(i, _):
        for b in (0, 1):
            ci = 2 * i + b
            nb = 1 - b
            for j in range(NGS):
                gat_copy(b, j).wait()
            @pl.when(ci >= 1)
            def _():
                out_copy(ci - 1, nb).wait()
            @pl.when(ci + 1 < n_chunks)
            def _():
                idx_copy(ci + 1, nb).wait()
                for j in range(NGS):
                    gat_copy(nb, j).start()
            @pl.when(ci + 2 < n_chunks)
            def _():
                idx_copy(ci + 2, b).start()

            rb = rows_v.at[b]

            @plsc.parallel_loop(0, CHUNK, step=1, unroll=4)
            def _(r):
                rb[r, pl.ds(0, 16)] = rb[r, pl.ds(0, 16)] + pos_v[r, pl.ds(0, 16)]
                rb[r, pl.ds(16, 16)] = rb[r, pl.ds(16, 16)] + pos_v[r, pl.ds(16, 16)]

            out_copy(ci, b).start()
        return _

    lax.fori_loop(0, n_chunks // 2, pair_body, None)
    out_copy(n_chunks - 1, (n_chunks - 1) % 2).wait()


def kernel(inputs, word_table, pos_table):
    b, s = inputs.shape
    n = b * s
    idx_flat = inputs.reshape(n).astype(jnp.int32)
    mesh = plsc.VectorSubcoreMesh(core_axis_name="c", subcore_axis_name="s")
    out = pl.kernel(
        _sc_body,
        out_type=jax.ShapeDtypeStruct((n, DIM), jnp.float32),
        mesh=mesh,
        compiler_params=pltpu.CompilerParams(use_tc_tiling_on_sc=False),
        scratch_types=[
            pltpu.VMEM((2, CHUNK), jnp.int32),
            pltpu.VMEM((2, CHUNK, DIM), jnp.float32),
            pltpu.VMEM((CHUNK, DIM), jnp.float32),
            pltpu.SemaphoreType.DMA((2,)),
            pltpu.SemaphoreType.DMA((2,)),
            pltpu.SemaphoreType.DMA((2,)),
        ],
    )(idx_flat, word_table, pos_table)
    return out.reshape(b, s, DIM)
